# baseline (speedup 1.0000x reference)
import functools

import jax
import jax.numpy as jnp
from jax import lax
from jax.experimental import pallas as pl
from jax.experimental.pallas import tpu as pltpu
from jax.experimental.pallas import tpu_sc as plsc

N = 6_400_000
N_STRUCTURES = 16384
N_SPECIES_PAD = 128
NC, NS = 2, 16
NW = NC * NS
CHUNK = N // NW
BLK = 10000
NBLK = CHUNK // BLK
L = 16


def _sc_body(le_hbm, z_hbm, b_hbm, scale_hbm, shift_hbm, out_hbm,
             scale_v, shift_v, le0_v, le1_v, z0_v, z1_v, b0_v, b1_v,
             acc_v, sem0, sem1):
    c = lax.axis_index("c")
    s = lax.axis_index("s")
    wid = s * NC + c
    base = wid * CHUNK

    pltpu.sync_copy(scale_hbm, scale_v)
    pltpu.sync_copy(shift_hbm, shift_v)

    zeros16 = jnp.zeros((L,), jnp.float32)

    def zero_body(i, carry):
        acc_v[pl.ds(i * L, L)] = zeros16
        return carry

    lax.fori_loop(0, N_STRUCTURES // L, zero_body, 0, unroll=8)

    bufs = ((le0_v, z0_v, b0_v, sem0), (le1_v, z1_v, b1_v, sem1))

    def start_fetch(g):
        le_b, z_b, b_b, sem = bufs[g % 2]
        off = base + g * BLK
        return (
            pltpu.async_copy(le_hbm.at[pl.ds(off, BLK)], le_b, sem),
            pltpu.async_copy(z_hbm.at[pl.ds(off, BLK)], z_b, sem),
            pltpu.async_copy(b_hbm.at[pl.ds(off, BLK)], b_b, sem),
        )

    fifteens = jnp.full((L,), L - 1, jnp.int32)

    def _splat(v, idx_vec):
        return lax.gather(
            v, idx_vec[:, None],
            lax.GatherDimensionNumbers(
                offset_dims=(), collapsed_slice_dims=(0,), start_index_map=(0,)),
            slice_sizes=(1,),
            mode=lax.GatherScatterMode.PROMISE_IN_BOUNDS)

    def compute_block(g, carry):
        le_b, z_b, b_b, _ = bufs[g % 2]

        def vec_body(j, carry2):
            run_sum, prev_last = carry2
            jl = j * L
            bb = b_b[pl.ds(jl, L)]
            zz = z_b[pl.ds(jl, L)]
            sc = plsc.load_gather(scale_v, [zz])
            sh = plsc.load_gather(shift_v, [zz])
            e = le_b[pl.ds(jl, L)] * sc + sh
            b_last = _splat(bb, fifteens)
            m_open = bb == prev_last
            fmask = b_last != prev_last
            plsc.addupdate_scatter(acc_v, [bb], e,
                                   mask=jnp.logical_not(m_open))
            flush = run_sum + jnp.where(m_open, e, 0.0)
            plsc.addupdate_scatter(acc_v, [prev_last], flush, mask=fmask)
            run_sum2 = jnp.where(fmask, 0.0, run_sum + e)
            return run_sum2, b_last

        return lax.fori_loop(0, BLK // L, vec_body, carry, unroll=5)

    descs = start_fetch(0)
    for d in descs:
        d.wait()
    prev_last0 = _splat(b0_v[pl.ds(0, L)], jnp.zeros((L,), jnp.int32))
    carry = (zeros16, prev_last0)
    pending = start_fetch(1)
    for g in range(NBLK):
        if g > 0:
            for d in pending:
                d.wait()
            if g + 1 < NBLK:
                pending = start_fetch(g + 1)
        carry = compute_block(g, carry)

    run_sum, prev_last = carry
    plsc.addupdate_scatter(acc_v, [prev_last], run_sum)

    pltpu.sync_copy(acc_v, out_hbm.at[wid])


@functools.partial(
    pl.kernel,
    out_type=jax.ShapeDtypeStruct((NW, N_STRUCTURES), jnp.float32),
    mesh=plsc.VectorSubcoreMesh(core_axis_name="c", subcore_axis_name="s"),
    scratch_types=[
        pltpu.VMEM((N_SPECIES_PAD,), jnp.float32),
        pltpu.VMEM((N_SPECIES_PAD,), jnp.float32),
        pltpu.VMEM((BLK,), jnp.float32),
        pltpu.VMEM((BLK,), jnp.float32),
        pltpu.VMEM((BLK,), jnp.int32),
        pltpu.VMEM((BLK,), jnp.int32),
        pltpu.VMEM((BLK,), jnp.int32),
        pltpu.VMEM((BLK,), jnp.int32),
        pltpu.VMEM((N_STRUCTURES,), jnp.float32),
        pltpu.SemaphoreType.DMA,
        pltpu.SemaphoreType.DMA,
    ],
    compiler_params=pltpu.CompilerParams(needs_layout_passes=False),
)
def _sc_partial_sums(*args):
    _sc_body(*args)


def _merge_body(parts_ref, out_ref):
    out_ref[...] = jnp.sum(parts_ref[...], axis=0)


def kernel(local_energies, Z, batch, scale, shift):
    scale_p = jnp.zeros((N_SPECIES_PAD,), jnp.float32).at[: scale.shape[0]].set(scale)
    shift_p = jnp.zeros((N_SPECIES_PAD,), jnp.float32).at[: shift.shape[0]].set(shift)
    parts = _sc_partial_sums(local_energies, Z, batch, scale_p, shift_p)
    total = pl.pallas_call(
        _merge_body,
        out_shape=jax.ShapeDtypeStruct((N_STRUCTURES,), jnp.float32),
    )(parts)
    return total


# speedup vs baseline: 1.0294x; 1.0294x over previous
import functools

import jax
import jax.numpy as jnp
from jax import lax
from jax.experimental import pallas as pl
from jax.experimental.pallas import tpu as pltpu
from jax.experimental.pallas import tpu_sc as plsc

N = 6_400_000
N_STRUCTURES = 16384
N_SPECIES_PAD = 128
NC, NS = 2, 16
NW = NC * NS
CHUNK = N // NW
BLK = 10000
NBLK = CHUNK // BLK
L = 16


def _sc_body(le_hbm, z_hbm, b_hbm, scale_hbm, shift_hbm, out_hbm,
             scale_v, shift_v, le0_v, le1_v, z0_v, z1_v, b0_v, b1_v,
             acc_v, sem0, sem1):
    c = lax.axis_index("c")
    s = lax.axis_index("s")
    wid = s * NC + c
    base = wid * CHUNK

    pltpu.sync_copy(scale_hbm, scale_v)
    pltpu.sync_copy(shift_hbm, shift_v)

    zeros16 = jnp.zeros((L,), jnp.float32)

    def zero_body(i, carry):
        acc_v[pl.ds(i * L, L)] = zeros16
        return carry

    lax.fori_loop(0, N_STRUCTURES // L, zero_body, 0, unroll=8)

    bufs = ((le0_v, z0_v, b0_v, sem0), (le1_v, z1_v, b1_v, sem1))

    def start_fetch(g):
        le_b, z_b, b_b, sem = bufs[g % 2]
        off = base + g * BLK
        return (
            pltpu.async_copy(le_hbm.at[pl.ds(off, BLK)], le_b, sem),
            pltpu.async_copy(z_hbm.at[pl.ds(off, BLK)], z_b, sem),
            pltpu.async_copy(b_hbm.at[pl.ds(off, BLK)], b_b, sem),
        )

    fifteens = jnp.full((L,), L - 1, jnp.int32)

    def _splat(v, idx_vec):
        return lax.gather(
            v, idx_vec[:, None],
            lax.GatherDimensionNumbers(
                offset_dims=(), collapsed_slice_dims=(0,), start_index_map=(0,)),
            slice_sizes=(1,),
            mode=lax.GatherScatterMode.PROMISE_IN_BOUNDS)

    def compute_block(g, carry):
        le_b, z_b, b_b, _ = bufs[g % 2]

        def vec_body(j, carry2):
            run_sum, prev_last = carry2
            jl = j * L
            bb = b_b[pl.ds(jl, L)]
            zz = z_b[pl.ds(jl, L)]
            sc = plsc.load_gather(scale_v, [zz])
            sh = plsc.load_gather(shift_v, [zz])
            e = le_b[pl.ds(jl, L)] * sc + sh
            b_last = _splat(bb, fifteens)
            m_open = bb == prev_last
            fmask = b_last != prev_last
            val = jnp.where(m_open, run_sum + e, e)
            idx = jnp.where(m_open, prev_last, bb)
            plsc.addupdate_scatter(acc_v, [idx], val, mask=fmask)
            run_sum2 = jnp.where(fmask, 0.0, run_sum + e)
            return run_sum2, b_last

        return lax.fori_loop(0, BLK // L, vec_body, carry, unroll=5)

    descs = start_fetch(0)
    for d in descs:
        d.wait()
    prev_last0 = _splat(b0_v[pl.ds(0, L)], jnp.zeros((L,), jnp.int32))
    carry = (zeros16, prev_last0)
    pending = start_fetch(1)
    for g in range(NBLK):
        if g > 0:
            for d in pending:
                d.wait()
            if g + 1 < NBLK:
                pending = start_fetch(g + 1)
        carry = compute_block(g, carry)

    run_sum, prev_last = carry
    plsc.addupdate_scatter(acc_v, [prev_last], run_sum)

    pltpu.sync_copy(acc_v, out_hbm.at[wid])


@functools.partial(
    pl.kernel,
    out_type=jax.ShapeDtypeStruct((NW, N_STRUCTURES), jnp.float32),
    mesh=plsc.VectorSubcoreMesh(core_axis_name="c", subcore_axis_name="s"),
    scratch_types=[
        pltpu.VMEM((N_SPECIES_PAD,), jnp.float32),
        pltpu.VMEM((N_SPECIES_PAD,), jnp.float32),
        pltpu.VMEM((BLK,), jnp.float32),
        pltpu.VMEM((BLK,), jnp.float32),
        pltpu.VMEM((BLK,), jnp.int32),
        pltpu.VMEM((BLK,), jnp.int32),
        pltpu.VMEM((BLK,), jnp.int32),
        pltpu.VMEM((BLK,), jnp.int32),
        pltpu.VMEM((N_STRUCTURES,), jnp.float32),
        pltpu.SemaphoreType.DMA,
        pltpu.SemaphoreType.DMA,
    ],
    compiler_params=pltpu.CompilerParams(needs_layout_passes=False),
)
def _sc_partial_sums(*args):
    _sc_body(*args)


def _merge_body(parts_ref, out_ref):
    out_ref[...] = jnp.sum(parts_ref[...], axis=0)


def kernel(local_energies, Z, batch, scale, shift):
    scale_p = jnp.zeros((N_SPECIES_PAD,), jnp.float32).at[: scale.shape[0]].set(scale)
    shift_p = jnp.zeros((N_SPECIES_PAD,), jnp.float32).at[: shift.shape[0]].set(shift)
    parts = _sc_partial_sums(local_energies, Z, batch, scale_p, shift_p)
    total = pl.pallas_call(
        _merge_body,
        out_shape=jax.ShapeDtypeStruct((N_STRUCTURES,), jnp.float32),
    )(parts)
    return total
